# baseline (speedup 1.0000x reference)
import functools

import jax
import jax.numpy as jnp
from jax import lax
from jax.experimental import pallas as pl
from jax.experimental.pallas import tpu as pltpu
from jax.experimental.pallas import tpu_sc as plsc

N_EDGES_K = 6400000
N_FRAMES_K = 16
NC = 2
NS = 16
L = 16
NW = NC * NS
PER_W = N_EDGES_K // NW
CHUNK = 20000
N_CHUNKS = PER_W // CHUNK


def _ev_body(rij_hbm, idx_hbm, cum_hbm, out_hbm, rij_v, idx_v, e_v, cum_v, scr_v):
    wid = lax.axis_index("c") * NS + lax.axis_index("s")
    base = wid * PER_W

    pltpu.sync_copy(cum_hbm, cum_v)
    cum_vec = cum_v[pl.ds(0, N_FRAMES_K)]
    cums = [cum_vec[f] for f in range(N_FRAMES_K - 1)]

    lane = lax.iota(jnp.int32, L)
    lane3 = lane * 3
    zero = jnp.zeros((L,), jnp.float32)

    def chunk_body(ch, accs):
        start = base + ch * CHUNK
        pltpu.sync_copy(rij_hbm.at[pl.ds(start * 3, CHUNK * 3)], rij_v)
        pltpu.sync_copy(idx_hbm.at[pl.ds(start, CHUNK)], idx_v)
        iv_first = idx_v[pl.ds(0, L)][0]
        iv_last = idx_v[pl.ds(CHUNK - L, L)][L - 1]

        def vec_body(i, cs):
            b = i * (3 * L) + lane3
            x = plsc.load_gather(rij_v, [b])
            y = plsc.load_gather(rij_v, [b + 1])
            z = plsc.load_gather(rij_v, [b + 2])
            d2 = x * x + y * y + z * z
            r = 1.0 / d2
            e = (r * r) * r
            e_v[pl.ds(i * L, L)] = e
            return cs + e

        cs = lax.fori_loop(0, CHUNK // L, vec_body, zero, unroll=4)

        new = []
        for f in range(N_FRAMES_K - 1):
            c = cums[f]
            def mid(acc=accs[f], c=c):
                def mbody(i, s):
                    idxv = idx_v[pl.ds(i * L, L)]
                    ev = e_v[pl.ds(i * L, L)]
                    return s + jnp.where(idxv < c, ev, 0.0)
                return acc + lax.fori_loop(0, CHUNK // L, mbody, zero, unroll=4)
            def triv(acc=accs[f], c=c):
                return acc + jnp.where(c > iv_last, cs, zero)
            new.append(lax.cond((c > iv_first) & (c <= iv_last), mid, triv))
        new.append(accs[N_FRAMES_K - 1] + cs)
        return tuple(new)

    accs = lax.fori_loop(0, N_CHUNKS, chunk_body,
                         tuple(zero for _ in range(N_FRAMES_K)))

    for f in range(N_FRAMES_K):
        scr_v[pl.ds(f * L, L)] = accs[f]
    col_idx = lane * L
    total = plsc.load_gather(scr_v, [col_idx])
    for c in range(1, L):
        total = total + plsc.load_gather(scr_v, [col_idx + c])
    scr_v[pl.ds(0, L)] = total
    pltpu.sync_copy(scr_v.at[pl.ds(0, L)], out_hbm.at[pl.ds(wid * N_FRAMES_K, N_FRAMES_K)])


@functools.partial(jax.jit, static_argnames=())
def kernel(Rij, idx_i, n_atoms):
    cum = jnp.cumsum(n_atoms, dtype=jnp.int32)
    rij_flat = Rij.reshape(-1)

    mesh = plsc.VectorSubcoreMesh(core_axis_name="c", subcore_axis_name="s")
    run = pl.kernel(
        _ev_body,
        out_type=jax.ShapeDtypeStruct((NW * N_FRAMES_K,), jnp.float32),
        mesh=mesh,
        scratch_types=[
            pltpu.VMEM((CHUNK * 3,), jnp.float32),
            pltpu.VMEM((CHUNK,), jnp.int32),
            pltpu.VMEM((CHUNK,), jnp.float32),
            pltpu.VMEM((N_FRAMES_K,), jnp.int32),
            pltpu.VMEM((N_FRAMES_K * L,), jnp.float32),
        ],
        compiler_params=pltpu.CompilerParams(needs_layout_passes=False),
    )
    rows = run(rij_flat, idx_i, cum).reshape(NW, N_FRAMES_K)
    prefix = rows.sum(axis=0)
    energy = prefix - jnp.concatenate([jnp.zeros((1,), jnp.float32), prefix[:-1]])
    return energy * 0.5


# speedup vs baseline: 1.9300x; 1.9300x over previous
import functools

import jax
import jax.numpy as jnp
from jax import lax
from jax.experimental import pallas as pl
from jax.experimental.pallas import tpu as pltpu
from jax.experimental.pallas import tpu_sc as plsc

N_EDGES_K = 6400000
N_FRAMES_K = 16
NC = 2
NS = 16
L = 16
NW = NC * NS
PER_W = N_EDGES_K // NW
CHUNK = 40000
N_CHUNKS = PER_W // CHUNK

BR = 25600
NB = N_EDGES_K // BR


def _energy_body(rij_ref, e_ref):
    blk = rij_ref[...]
    d2 = jnp.sum(blk * blk, axis=1)
    r = 1.0 / d2
    e_ref[...] = (r * r) * r


def _segment_body(e_hbm, idx_hbm, cum_hbm, out_hbm, e_v, idx_v, cum_v, scr_v):
    wid = lax.axis_index("c") * NS + lax.axis_index("s")
    base = wid * PER_W

    pltpu.sync_copy(cum_hbm, cum_v)
    cum_vec = cum_v[pl.ds(0, N_FRAMES_K)]
    cums = [cum_vec[f] for f in range(N_FRAMES_K - 1)]

    lane = lax.iota(jnp.int32, L)
    zero = jnp.zeros((L,), jnp.float32)

    def chunk_body(ch, accs):
        start = base + ch * CHUNK
        pltpu.sync_copy(e_hbm.at[pl.ds(start, CHUNK)], e_v)
        pltpu.sync_copy(idx_hbm.at[pl.ds(start, CHUNK)], idx_v)
        iv_first = idx_v[pl.ds(0, L)][0]
        iv_last = idx_v[pl.ds(CHUNK - L, L)][L - 1]

        def vec_body(i, cs):
            return cs + e_v[pl.ds(i * L, L)]

        cs = lax.fori_loop(0, CHUNK // L, vec_body, zero, unroll=8)

        new = []
        for f in range(N_FRAMES_K - 1):
            c = cums[f]
            def mid(acc=accs[f], c=c):
                def mbody(i, s):
                    idxv = idx_v[pl.ds(i * L, L)]
                    ev = e_v[pl.ds(i * L, L)]
                    return s + jnp.where(idxv < c, ev, 0.0)
                return acc + lax.fori_loop(0, CHUNK // L, mbody, zero, unroll=4)
            def triv(acc=accs[f], c=c):
                return acc + jnp.where(c > iv_last, cs, zero)
            new.append(lax.cond((c > iv_first) & (c <= iv_last), mid, triv))
        new.append(accs[N_FRAMES_K - 1] + cs)
        return tuple(new)

    accs = lax.fori_loop(0, N_CHUNKS, chunk_body,
                         tuple(zero for _ in range(N_FRAMES_K)))

    for f in range(N_FRAMES_K):
        scr_v[pl.ds(f * L, L)] = accs[f]
    col_idx = lane * L
    total = plsc.load_gather(scr_v, [col_idx])
    for c in range(1, L):
        total = total + plsc.load_gather(scr_v, [col_idx + c])
    scr_v[pl.ds(0, L)] = total
    pltpu.sync_copy(scr_v.at[pl.ds(0, L)],
                    out_hbm.at[pl.ds(wid * N_FRAMES_K, N_FRAMES_K)])


@functools.partial(jax.jit, static_argnames=())
def kernel(Rij, idx_i, n_atoms):
    cum = jnp.cumsum(n_atoms, dtype=jnp.int32)

    e = pl.pallas_call(
        _energy_body,
        grid=(NB,),
        in_specs=[pl.BlockSpec((BR, 3), lambda i: (i, 0))],
        out_specs=pl.BlockSpec((BR,), lambda i: (i,)),
        out_shape=jax.ShapeDtypeStruct((N_EDGES_K,), jnp.float32),
    )(Rij)

    mesh = plsc.VectorSubcoreMesh(core_axis_name="c", subcore_axis_name="s")
    run = pl.kernel(
        _segment_body,
        out_type=jax.ShapeDtypeStruct((NW * N_FRAMES_K,), jnp.float32),
        mesh=mesh,
        scratch_types=[
            pltpu.VMEM((CHUNK,), jnp.float32),
            pltpu.VMEM((CHUNK,), jnp.int32),
            pltpu.VMEM((N_FRAMES_K,), jnp.int32),
            pltpu.VMEM((N_FRAMES_K * L,), jnp.float32),
        ],
        compiler_params=pltpu.CompilerParams(needs_layout_passes=False),
    )
    rows = run(e, idx_i, cum).reshape(NW, N_FRAMES_K)
    prefix = rows.sum(axis=0)
    energy = prefix - jnp.concatenate([jnp.zeros((1,), jnp.float32), prefix[:-1]])
    return energy * 0.5


# speedup vs baseline: 3.5237x; 1.8258x over previous
import functools

import jax
import jax.numpy as jnp
from jax import lax
from jax.experimental import pallas as pl
from jax.experimental.pallas import tpu as pltpu
from jax.experimental.pallas import tpu_sc as plsc

N_EDGES_K = 6400000
N_FRAMES_K = 16
NC = 2
NS = 16
L = 16
NW = NC * NS
PER_W = N_EDGES_K // NW
CHUNK = 40000
N_CHUNKS = PER_W // CHUNK

BR = 10240
NB = N_EDGES_K // BR


DEPTH = 4


def _energy_body(rij_hbm, e_ref, buf, sem):
    g = pl.program_id(0)

    @pl.when(g == 0)
    def _prime():
        for s in range(DEPTH - 1):
            pltpu.make_async_copy(rij_hbm.at[pl.ds(s * BR, BR), :],
                                  buf.at[s], sem.at[s]).start()

    nxt = g + DEPTH - 1
    @pl.when(nxt < NB)
    def _prefetch():
        pltpu.make_async_copy(rij_hbm.at[pl.ds(nxt * BR, BR), :],
                              buf.at[nxt % DEPTH], sem.at[nxt % DEPTH]).start()

    slot = g % DEPTH
    pltpu.make_async_copy(rij_hbm.at[pl.ds(g * BR, BR), :],
                          buf.at[slot], sem.at[slot]).wait()
    blk = buf[slot]
    sq = blk * blk
    d2 = jax.lax.dot_general(jnp.ones((1, 3), jnp.float32), sq,
                             (((1,), (1,)), ((), ())),
                             preferred_element_type=jnp.float32)
    r = 1.0 / d2
    e_ref[...] = ((r * r) * r).reshape(BR // 128, 128)


def _segment_body(e_hbm, idx_hbm, cum_hbm, out_hbm, e_v, idx_v, cum_v, scr_v):
    wid = lax.axis_index("c") * NS + lax.axis_index("s")
    base = wid * PER_W

    pltpu.sync_copy(cum_hbm, cum_v)
    cum_vec = cum_v[pl.ds(0, N_FRAMES_K)]
    cums = [cum_vec[f] for f in range(N_FRAMES_K - 1)]

    lane = lax.iota(jnp.int32, L)
    zero = jnp.zeros((L,), jnp.float32)

    def chunk_body(ch, accs):
        start = base + ch * CHUNK
        pltpu.sync_copy(e_hbm.at[pl.ds(start, CHUNK)], e_v)
        pltpu.sync_copy(idx_hbm.at[pl.ds(start, CHUNK)], idx_v)
        iv_first = idx_v[pl.ds(0, L)][0]
        iv_last = idx_v[pl.ds(CHUNK - L, L)][L - 1]

        def vec_body(i, cs):
            return cs + e_v[pl.ds(i * L, L)]

        cs = lax.fori_loop(0, CHUNK // L, vec_body, zero, unroll=8)

        new = []
        for f in range(N_FRAMES_K - 1):
            c = cums[f]
            def mid(acc=accs[f], c=c):
                def mbody(i, s):
                    idxv = idx_v[pl.ds(i * L, L)]
                    ev = e_v[pl.ds(i * L, L)]
                    return s + jnp.where(idxv < c, ev, 0.0)
                return acc + lax.fori_loop(0, CHUNK // L, mbody, zero, unroll=4)
            def triv(acc=accs[f], c=c):
                return acc + jnp.where(c > iv_last, cs, zero)
            new.append(lax.cond((c > iv_first) & (c <= iv_last), mid, triv))
        new.append(accs[N_FRAMES_K - 1] + cs)
        return tuple(new)

    accs = lax.fori_loop(0, N_CHUNKS, chunk_body,
                         tuple(zero for _ in range(N_FRAMES_K)))

    for f in range(N_FRAMES_K):
        scr_v[pl.ds(f * L, L)] = accs[f]
    col_idx = lane * L
    total = plsc.load_gather(scr_v, [col_idx])
    for c in range(1, L):
        total = total + plsc.load_gather(scr_v, [col_idx + c])
    scr_v[pl.ds(0, L)] = total
    pltpu.sync_copy(scr_v.at[pl.ds(0, L)],
                    out_hbm.at[pl.ds(wid * N_FRAMES_K, N_FRAMES_K)])


@functools.partial(jax.jit, static_argnames=())
def kernel(Rij, idx_i, n_atoms):
    cum = jnp.cumsum(n_atoms, dtype=jnp.int32)

    e = pl.pallas_call(
        _energy_body,
        grid=(NB,),
        in_specs=[pl.BlockSpec(memory_space=pl.ANY)],
        out_specs=pl.BlockSpec((BR // 128, 128), lambda i: (i, 0)),
        out_shape=jax.ShapeDtypeStruct((N_EDGES_K // 128, 128), jnp.float32),
        scratch_shapes=[
            pltpu.VMEM((DEPTH, BR, 3), jnp.float32),
            pltpu.SemaphoreType.DMA((DEPTH,)),
        ],
    )(Rij)
    e = e.reshape(N_EDGES_K)

    mesh = plsc.VectorSubcoreMesh(core_axis_name="c", subcore_axis_name="s")
    run = pl.kernel(
        _segment_body,
        out_type=jax.ShapeDtypeStruct((NW * N_FRAMES_K,), jnp.float32),
        mesh=mesh,
        scratch_types=[
            pltpu.VMEM((CHUNK,), jnp.float32),
            pltpu.VMEM((CHUNK,), jnp.int32),
            pltpu.VMEM((N_FRAMES_K,), jnp.int32),
            pltpu.VMEM((N_FRAMES_K * L,), jnp.float32),
        ],
        compiler_params=pltpu.CompilerParams(needs_layout_passes=False),
    )
    rows = run(e, idx_i, cum).reshape(NW, N_FRAMES_K)
    prefix = rows.sum(axis=0)
    energy = prefix - jnp.concatenate([jnp.zeros((1,), jnp.float32), prefix[:-1]])
    return energy * 0.5
